# baseline (speedup 1.0000x reference)
import functools

import jax
import jax.numpy as jnp
from jax import lax
from jax.experimental import pallas as pl
from jax.experimental.pallas import tpu as pltpu
from jax.experimental.pallas import tpu_sc as plsc

VOCAB = 100000
BATCH = 16384
LANES = 16
NW = 32
B_PER_T = BATCH // 16
N_CHUNK = B_PER_T // 128
V_SH = 102400
Z_PER_S = V_SH // 16
V_SLICE = 3200
V_LB = 96768
V_PREF = 31 * V_SLICE - V_LB
V_TAIL = VOCAB - V_LB - V_SLICE
NT = (5, 10, 6)
NTOT = sum(NT)
NR = 9
NBUF = 3

_mesh = plsc.VectorSubcoreMesh(core_axis_name="c", subcore_axis_name="s")


@functools.partial(
    pl.kernel,
    mesh=_mesh,
    out_type=jax.ShapeDtypeStruct((NW * NR * LANES,), jnp.float32),
    scratch_types=[
        pltpu.VMEM((16, N_CHUNK, 128), jnp.int32),
        pltpu.VMEM((128,), jnp.float32),
        pltpu.VMEM((Z_PER_S,), jnp.float32),
        pltpu.VMEM((V_SLICE,), jnp.float32),
        pltpu.VMEM((NBUF, 3, V_SLICE), jnp.float32),
        pltpu.VMEM((NR * LANES,), jnp.float32),
        pltpu.VMEM((NTOT * 3 * V_TAIL,), jnp.float32),
        pltpu.VMEM_SHARED((V_SH,), jnp.float32),
        pltpu.SemaphoreType.DMA,
        pltpu.SemaphoreType.DMA,
        pltpu.SemaphoreType.DMA,
    ],
    compiler_params=pltpu.CompilerParams(needs_layout_passes=False),
)
def _sc_fused(idx_hbm, t0_hbm, t1_hbm, t2_hbm, tail_hbm, out_hbm,
              idx_v, ones_v, zero_v, cnt_v, tbuf_v, acc_v, tail_v, cnt_sh,
              sem0, sem1, sem2):
    cid = lax.axis_index("c")
    sid = lax.axis_index("s")
    wid = sid * 2 + cid
    last = wid == NW - 1
    vb = jnp.where(last, V_LB, wid * V_SLICE)
    sems = (sem0, sem1, sem2)
    tabs = ([(t0_hbm, t) for t in range(NT[0])]
            + [(t1_hbm, t) for t in range(NT[1])]
            + [(t2_hbm, t) for t in range(NT[2])])

    def _issue(i):
        tref, t = tabs[i]
        return pltpu.async_copy(tref.at[t, :, pl.ds(vb, V_SLICE)],
                                tbuf_v.at[i % NBUF], sems[i % NBUF])

    handles = {i: _issue(i) for i in range(NBUF)}

    pltpu.sync_copy(idx_hbm, idx_v)
    pltpu.sync_copy(tail_hbm, tail_v)

    ones16 = jnp.ones((LANES,), jnp.float32)
    zeros16 = jnp.zeros((LANES,), jnp.float32)
    for i in range(128 // LANES):
        ones_v[pl.ds(i * LANES, LANES)] = ones16

    def _zero(i, carry):
        zero_v[pl.ds(i * LANES, LANES)] = zeros16
        return carry

    lax.fori_loop(0, Z_PER_S // LANES, _zero, 0)
    pltpu.sync_copy(zero_v, cnt_sh.at[pl.ds(sid * Z_PER_S, Z_PER_S)])
    plsc.subcore_barrier()

    for j in range(N_CHUNK):
        pltpu.sync_copy(ones_v, cnt_sh.at[idx_v.at[sid, j]], add=True)
    plsc.subcore_barrier()

    pltpu.sync_copy(cnt_sh.at[pl.ds(vb, V_SLICE)], cnt_v)

    @pl.when(last)
    def _():
        def _zp(i, carry):
            cnt_v[pl.ds(i * LANES, LANES)] = zeros16
            return carry
        lax.fori_loop(0, V_PREF // LANES, _zp, 0)

    pltpu.sync_copy(cnt_sh.at[pl.ds(V_LB + V_SLICE, V_TAIL)],
                    ones_v.at[pl.ds(0, V_TAIL)])
    lastf = jnp.where(last, ones16, zeros16)
    ct0 = ones_v[pl.ds(0, LANES)] * lastf
    ct1 = ones_v[pl.ds(LANES, LANES)] * lastf

    accs = []
    a0 = a1 = a2 = jnp.zeros((LANES,), jnp.float32)
    for i in range(NTOT):
        slot = i % NBUF
        handles[i].wait()

        def _mac(k, carry, slot=slot):
            b0, b1, b2 = carry
            for u in range(4):
                o = (k * 4 + u) * LANES
                c16 = cnt_v[pl.ds(o, LANES)]
                b0 = b0 + tbuf_v[slot, 0, pl.ds(o, LANES)] * c16
                b1 = b1 + tbuf_v[slot, 1, pl.ds(o, LANES)] * c16
                b2 = b2 + tbuf_v[slot, 2, pl.ds(o, LANES)] * c16
            return b0, b1, b2

        r0, r1, r2 = lax.fori_loop(0, V_SLICE // (4 * LANES), _mac,
                                   (jnp.zeros((LANES,), jnp.float32),) * 3)
        base = i * 3 * V_TAIL
        r0 = r0 + tail_v[pl.ds(base, LANES)] * ct0
        r0 = r0 + tail_v[pl.ds(base + LANES, LANES)] * ct1
        r1 = r1 + tail_v[pl.ds(base + V_TAIL, LANES)] * ct0
        r1 = r1 + tail_v[pl.ds(base + V_TAIL + LANES, LANES)] * ct1
        r2 = r2 + tail_v[pl.ds(base + 2 * V_TAIL, LANES)] * ct0
        r2 = r2 + tail_v[pl.ds(base + 2 * V_TAIL + LANES, LANES)] * ct1
        a0, a1, a2 = a0 + r0, a1 + r1, a2 + r2

        if i + NBUF < NTOT:
            handles[i + NBUF] = _issue(i + NBUF)
        if i in (NT[0] - 1, NT[0] + NT[1] - 1, NTOT - 1):
            accs += [a0, a1, a2]
            a0 = a1 = a2 = jnp.zeros((LANES,), jnp.float32)

    for a in range(NR):
        acc_v[pl.ds(a * LANES, LANES)] = accs[a]
    pltpu.sync_copy(acc_v, out_hbm.at[pl.ds(wid * NR * LANES, NR * LANES)])


def _tc_fold_body(p_ref, out_ref):
    n = NW * NR
    rs = jnp.sum(p_ref[...], axis=1, keepdims=True)
    j3 = lax.broadcasted_iota(jnp.int32, (n, 3), 0) % NR
    d3 = lax.broadcasted_iota(jnp.int32, (n, 3), 1)
    cmat = jnp.where(j3 % 3 == d3, rs, 0.0)
    jj = lax.broadcasted_iota(jnp.int32, (3, n), 1) % NR
    gg = lax.broadcasted_iota(jnp.int32, (3, n), 0)
    amat = (jj // 3 == gg).astype(jnp.float32)
    out_ref[...] = jnp.dot(amat, cmat, preferred_element_type=jnp.float32)


def _tc_fold(partials):
    return pl.pallas_call(
        _tc_fold_body,
        out_shape=jax.ShapeDtypeStruct((3, 3), jnp.float32),
    )(partials)


def kernel(e_input, tables0, tables1, tables2):
    idx = e_input.astype(jnp.int32).reshape(16, N_CHUNK, 128)
    x0 = jnp.transpose(tables0, (0, 2, 1))
    x1 = jnp.transpose(tables1, (0, 2, 1))
    x2 = jnp.transpose(tables2, (0, 2, 1))
    vt = V_LB + V_SLICE
    tail = jnp.concatenate([tables0[:, vt:, :], tables1[:, vt:, :],
                            tables2[:, vt:, :]], axis=0)
    tailx = jnp.transpose(tail, (0, 2, 1)).reshape(NTOT * 3 * V_TAIL)
    partials = _sc_fused(idx, x0, x1, x2, tailx).reshape(NW * NR, LANES)
    return _tc_fold(partials)


# speedup vs baseline: 1.0013x; 1.0013x over previous
import functools

import jax
import jax.numpy as jnp
from jax import lax
from jax.experimental import pallas as pl
from jax.experimental.pallas import tpu as pltpu
from jax.experimental.pallas import tpu_sc as plsc

VOCAB = 100000
BATCH = 16384
LANES = 16
NW = 32
B_PER_T = BATCH // 16
N_CHUNK = B_PER_T // 128
V_SH = 102400
Z_PER_S = V_SH // 16
V_SLICE = 3200
V_LB = 96768
V_PREF = 31 * V_SLICE - V_LB
V_TAIL = VOCAB - V_LB - V_SLICE
NT = (5, 10, 6)
NTOT = sum(NT)
NR = 9
NBUF = 6

_mesh = plsc.VectorSubcoreMesh(core_axis_name="c", subcore_axis_name="s")


@functools.partial(
    pl.kernel,
    mesh=_mesh,
    out_type=jax.ShapeDtypeStruct((NW * NR * LANES,), jnp.float32),
    scratch_types=[
        pltpu.VMEM((16, N_CHUNK, 128), jnp.int32),
        pltpu.VMEM((128,), jnp.float32),
        pltpu.VMEM((Z_PER_S,), jnp.float32),
        pltpu.VMEM((V_SLICE,), jnp.float32),
        pltpu.VMEM((NBUF, 3, V_SLICE), jnp.float32),
        pltpu.VMEM((NR * LANES,), jnp.float32),
        pltpu.VMEM((NTOT * 3 * V_TAIL,), jnp.float32),
        pltpu.VMEM_SHARED((V_SH,), jnp.float32),
        pltpu.SemaphoreType.DMA,
        pltpu.SemaphoreType.DMA,
        pltpu.SemaphoreType.DMA,
        pltpu.SemaphoreType.DMA,
        pltpu.SemaphoreType.DMA,
        pltpu.SemaphoreType.DMA,
    ],
    compiler_params=pltpu.CompilerParams(needs_layout_passes=False),
)
def _sc_fused(idx_hbm, t0_hbm, t1_hbm, t2_hbm, tail_hbm, out_hbm,
              idx_v, ones_v, zero_v, cnt_v, tbuf_v, acc_v, tail_v, cnt_sh,
              sem0, sem1, sem2, sem3, sem4, sem5):
    cid = lax.axis_index("c")
    sid = lax.axis_index("s")
    wid = sid * 2 + cid
    last = wid == NW - 1
    vb = jnp.where(last, V_LB, wid * V_SLICE)
    sems = (sem0, sem1, sem2, sem3, sem4, sem5)
    tabs = ([(t0_hbm, t) for t in range(NT[0])]
            + [(t1_hbm, t) for t in range(NT[1])]
            + [(t2_hbm, t) for t in range(NT[2])])

    def _issue(i):
        tref, t = tabs[i]
        return pltpu.async_copy(tref.at[t, :, pl.ds(vb, V_SLICE)],
                                tbuf_v.at[i % NBUF], sems[i % NBUF])

    handles = {i: _issue(i) for i in range(NBUF)}

    pltpu.sync_copy(idx_hbm, idx_v)
    pltpu.sync_copy(tail_hbm, tail_v)

    ones16 = jnp.ones((LANES,), jnp.float32)
    zeros16 = jnp.zeros((LANES,), jnp.float32)
    for i in range(128 // LANES):
        ones_v[pl.ds(i * LANES, LANES)] = ones16

    def _zero(i, carry):
        zero_v[pl.ds(i * LANES, LANES)] = zeros16
        return carry

    lax.fori_loop(0, Z_PER_S // LANES, _zero, 0)
    pltpu.sync_copy(zero_v, cnt_sh.at[pl.ds(sid * Z_PER_S, Z_PER_S)])
    plsc.subcore_barrier()

    for j in range(N_CHUNK):
        pltpu.sync_copy(ones_v, cnt_sh.at[idx_v.at[sid, j]], add=True)
    plsc.subcore_barrier()

    pltpu.sync_copy(cnt_sh.at[pl.ds(vb, V_SLICE)], cnt_v)

    @pl.when(last)
    def _():
        def _zp(i, carry):
            cnt_v[pl.ds(i * LANES, LANES)] = zeros16
            return carry
        lax.fori_loop(0, V_PREF // LANES, _zp, 0)

    pltpu.sync_copy(cnt_sh.at[pl.ds(V_LB + V_SLICE, V_TAIL)],
                    ones_v.at[pl.ds(0, V_TAIL)])
    lastf = jnp.where(last, ones16, zeros16)
    ct0 = ones_v[pl.ds(0, LANES)] * lastf
    ct1 = ones_v[pl.ds(LANES, LANES)] * lastf

    accs = []
    a0 = a1 = a2 = jnp.zeros((LANES,), jnp.float32)
    for i in range(NTOT):
        slot = i % NBUF
        handles[i].wait()

        def _mac(k, carry, slot=slot):
            b0, b1, b2 = carry
            for u in range(8):
                o = (k * 8 + u) * LANES
                c16 = cnt_v[pl.ds(o, LANES)]
                b0 = b0 + tbuf_v[slot, 0, pl.ds(o, LANES)] * c16
                b1 = b1 + tbuf_v[slot, 1, pl.ds(o, LANES)] * c16
                b2 = b2 + tbuf_v[slot, 2, pl.ds(o, LANES)] * c16
            return b0, b1, b2

        r0, r1, r2 = lax.fori_loop(0, V_SLICE // (8 * LANES), _mac,
                                   (jnp.zeros((LANES,), jnp.float32),) * 3)
        base = i * 3 * V_TAIL
        r0 = r0 + tail_v[pl.ds(base, LANES)] * ct0
        r0 = r0 + tail_v[pl.ds(base + LANES, LANES)] * ct1
        r1 = r1 + tail_v[pl.ds(base + V_TAIL, LANES)] * ct0
        r1 = r1 + tail_v[pl.ds(base + V_TAIL + LANES, LANES)] * ct1
        r2 = r2 + tail_v[pl.ds(base + 2 * V_TAIL, LANES)] * ct0
        r2 = r2 + tail_v[pl.ds(base + 2 * V_TAIL + LANES, LANES)] * ct1
        a0, a1, a2 = a0 + r0, a1 + r1, a2 + r2

        if i + NBUF < NTOT:
            handles[i + NBUF] = _issue(i + NBUF)
        if i in (NT[0] - 1, NT[0] + NT[1] - 1, NTOT - 1):
            accs += [a0, a1, a2]
            a0 = a1 = a2 = jnp.zeros((LANES,), jnp.float32)

    for a in range(NR):
        acc_v[pl.ds(a * LANES, LANES)] = accs[a]
    pltpu.sync_copy(acc_v, out_hbm.at[pl.ds(wid * NR * LANES, NR * LANES)])


def _tc_fold_body(p_ref, out_ref):
    n = NW * NR
    rs = jnp.sum(p_ref[...], axis=1, keepdims=True)
    j3 = lax.broadcasted_iota(jnp.int32, (n, 3), 0) % NR
    d3 = lax.broadcasted_iota(jnp.int32, (n, 3), 1)
    cmat = jnp.where(j3 % 3 == d3, rs, 0.0)
    jj = lax.broadcasted_iota(jnp.int32, (3, n), 1) % NR
    gg = lax.broadcasted_iota(jnp.int32, (3, n), 0)
    amat = (jj // 3 == gg).astype(jnp.float32)
    out_ref[...] = jnp.dot(amat, cmat, preferred_element_type=jnp.float32)


def _tc_fold(partials):
    return pl.pallas_call(
        _tc_fold_body,
        out_shape=jax.ShapeDtypeStruct((3, 3), jnp.float32),
    )(partials)


def kernel(e_input, tables0, tables1, tables2):
    idx = e_input.astype(jnp.int32).reshape(16, N_CHUNK, 128)
    x0 = jnp.transpose(tables0, (0, 2, 1))
    x1 = jnp.transpose(tables1, (0, 2, 1))
    x2 = jnp.transpose(tables2, (0, 2, 1))
    vt = V_LB + V_SLICE
    tail = jnp.concatenate([tables0[:, vt:, :], tables1[:, vt:, :],
                            tables2[:, vt:, :]], axis=0)
    tailx = jnp.transpose(tail, (0, 2, 1)).reshape(NTOT * 3 * V_TAIL)
    partials = _sc_fused(idx, x0, x1, x2, tailx).reshape(NW * NR, LANES)
    return _tc_fold(partials)
